# Initial kernel scaffold; baseline (speedup 1.0000x reference)
import jax
import jax.numpy as jnp
from jax.experimental import pallas as pl


def kernel(q_tensor, k_tensor, v_tensor):
    raise NotImplementedError("write your pallas kernel here")


# speedup vs baseline: 1.0454x; 1.0454x over previous
import functools
import math

import jax
import jax.numpy as jnp
from jax import lax
from jax.experimental import pallas as pl
from jax.experimental.pallas import tpu as pltpu

B, H, S, D = 1, 16, 2048, 128
N_ELEM = B * H * S * D
N_OUT = int(0.005 * N_ELEM)
SINK_LENGTH = 4
QMAX = 127.0
BLK_Q = 512


def _scale_kernel(k_ref, v_ref, thr_ref, kmax_ref, vmax_ref):
    h = pl.program_id(0)
    kabs = jnp.abs(k_ref[0, 0])
    vabs = jnp.abs(v_ref[0, 0])
    t_k = thr_ref[0:1, :]
    t_v = thr_ref[1:2, :]
    km = jnp.where(kabs < t_k, kabs, 0.0)
    rows = lax.broadcasted_iota(jnp.int32, (S, 1), 0)
    vmask = (vabs < t_v) & (rows >= SINK_LENGTH)
    vm = jnp.where(vmask, vabs, 0.0)
    kblk = jnp.max(km, axis=0, keepdims=True)
    vblk = jnp.max(vm, axis=0, keepdims=True)

    @pl.when(h == 0)
    def _():
        kmax_ref[...] = jnp.zeros_like(kmax_ref)
        vmax_ref[...] = jnp.zeros_like(vmax_ref)

    kmax_ref[...] = jnp.maximum(kmax_ref[...], kblk)
    vmax_ref[...] = jnp.maximum(vmax_ref[...], vblk)


def _masked_absmax(k, v, thr):
    out = pl.pallas_call(
        _scale_kernel,
        grid=(H,),
        in_specs=[
            pl.BlockSpec((1, 1, S, D), lambda h: (0, h, 0, 0)),
            pl.BlockSpec((1, 1, S, D), lambda h: (0, h, 0, 0)),
            pl.BlockSpec((8, D), lambda h: (0, 0)),
        ],
        out_specs=[
            pl.BlockSpec((1, D), lambda h: (0, 0)),
            pl.BlockSpec((1, D), lambda h: (0, 0)),
        ],
        out_shape=[
            jax.ShapeDtypeStruct((1, D), jnp.float32),
            jax.ShapeDtypeStruct((1, D), jnp.float32),
        ],
    )(k, v, thr)
    return out


def _flash_kernel(params_ref, q_ref, k_ref, v_ref, o_ref, krec_ref, vrec_ref):
    qb = pl.program_id(1)

    @pl.when(qb == 0)
    def _():
        kraw = k_ref[0, 0]
        vraw = v_ref[0, 0]
        ks = params_ref[0:1, :]
        vs = params_ref[1:2, :]
        t_k = params_ref[2:3, :]
        t_v = params_ref[3:4, :]
        kdq = jnp.clip(jnp.round(kraw / ks), -QMAX, QMAX) * ks
        krec_ref[...] = jnp.where(jnp.abs(kraw) >= t_k, kraw, kdq)
        vdq = jnp.clip(jnp.round(vraw / vs), -QMAX, QMAX) * vs
        rows = lax.broadcasted_iota(jnp.int32, (S, 1), 0)
        keep = (jnp.abs(vraw) >= t_v) | (rows < SINK_LENGTH)
        vrec_ref[...] = jnp.where(keep, vraw, vdq)

    qblk = q_ref[0, 0]
    s = lax.dot_general(
        qblk, krec_ref[...], (((1,), (1,)), ((), ())),
        preferred_element_type=jnp.float32,
        precision=lax.Precision.HIGHEST,
    ) / math.sqrt(float(D))
    m = jnp.max(s, axis=-1, keepdims=True)
    p = jnp.exp(s - m)
    denom = jnp.sum(p, axis=-1, keepdims=True)
    o = lax.dot_general(
        p, vrec_ref[...], (((1,), (0,)), ((), ())),
        preferred_element_type=jnp.float32,
        precision=lax.Precision.HIGHEST,
    )
    o_ref[0, 0] = o / denom


def _attention(params, q, k, v):
    return pl.pallas_call(
        _flash_kernel,
        grid=(H, S // BLK_Q),
        in_specs=[
            pl.BlockSpec((8, D), lambda h, qb: (0, 0)),
            pl.BlockSpec((1, 1, BLK_Q, D), lambda h, qb: (0, h, qb, 0)),
            pl.BlockSpec((1, 1, S, D), lambda h, qb: (0, h, 0, 0)),
            pl.BlockSpec((1, 1, S, D), lambda h, qb: (0, h, 0, 0)),
        ],
        out_specs=pl.BlockSpec((1, 1, BLK_Q, D), lambda h, qb: (0, h, qb, 0)),
        out_shape=jax.ShapeDtypeStruct((B, H, S, D), jnp.float32),
        scratch_shapes=[
            pltpu.VMEM((S, D), jnp.float32),
            pltpu.VMEM((S, D), jnp.float32),
        ],
    )(params, q, k, v)


def kernel(q_tensor, k_tensor, v_tensor):
    kabs = jnp.abs(k_tensor).reshape(-1)
    vabs = jnp.abs(v_tensor).reshape(-1)
    t_k = lax.top_k(kabs, N_OUT)[0][-1]
    t_v = lax.top_k(vabs, N_OUT)[0][-1]

    thr = jnp.zeros((8, D), jnp.float32)
    thr = thr.at[0, :].set(t_k)
    thr = thr.at[1, :].set(t_v)

    kmax, vmax = _masked_absmax(k_tensor, v_tensor, thr)
    k_scale = jnp.maximum(kmax[0], 1e-6) / QMAX
    v_scale = jnp.maximum(vmax[0], 1e-6) / QMAX

    params = jnp.zeros((8, D), jnp.float32)
    params = params.at[0, :].set(k_scale)
    params = params.at[1, :].set(v_scale)
    params = params.at[2, :].set(t_k)
    params = params.at[3, :].set(t_v)

    return _attention(params, q_tensor, k_tensor, v_tensor)


# speedup vs baseline: 8.0478x; 7.6985x over previous
import functools
import math

import jax
import jax.numpy as jnp
from jax import lax
from jax.experimental import pallas as pl
from jax.experimental.pallas import tpu as pltpu
from jax.experimental.pallas import tpu_sc as plsc

B, H, S, D = 1, 16, 2048, 128
N_ELEM = B * H * S * D
N_OUT = int(0.005 * N_ELEM)
SINK_LENGTH = 4
QMAX = 127.0
BLK_Q = 512


_SC_NC, _SC_NS, _SC_L = 2, 16, 16
_NW = _SC_NC * _SC_NS
_PER_W = N_ELEM // _NW
_CHUNK = 8192
_NB = 4096

_sc_mesh = plsc.VectorSubcoreMesh(core_axis_name="c", subcore_axis_name="s")


@functools.partial(
    pl.kernel,
    mesh=_sc_mesh,
    out_type=[
        jax.ShapeDtypeStruct((_NW * _NB,), jnp.int32),
        jax.ShapeDtypeStruct((_NW * _NB,), jnp.int32),
    ],
    scratch_types=[
        pltpu.VMEM((128,), jnp.int32),
        pltpu.VMEM((_CHUNK,), jnp.int32),
        pltpu.VMEM((_SC_L * _NB,), jnp.int32),
        pltpu.VMEM((_NB,), jnp.int32),
    ],
    compiler_params=pltpu.CompilerParams(needs_layout_passes=False),
)
def _sc_hist(k_hbm, v_hbm, par_hbm, outk_hbm, outv_hbm,
             par_v, buf_v, hist_v, res_v):
    wid = lax.axis_index("s") * _SC_NC + lax.axis_index("c")
    base = wid * _PER_W
    pltpu.sync_copy(par_hbm, par_v)
    lanes = lax.iota(jnp.int32, _SC_L)
    ones = jnp.ones((_SC_L,), jnp.int32)
    zeros16 = jnp.zeros((_SC_L,), jnp.int32)
    signmask = jnp.full((_SC_L,), 0x7FFFFFFF, jnp.int32)

    for t, (data_hbm, out_hbm) in enumerate(((k_hbm, outk_hbm),
                                             (v_hbm, outv_hbm))):
        fs = par_v[pl.ds((4 * t + 0) * 16, 16)]
        fv = par_v[pl.ds((4 * t + 1) * 16, 16)]
        bs = par_v[pl.ds((4 * t + 2) * 16, 16)]
        bm = par_v[pl.ds((4 * t + 3) * 16, 16)]

        def _zero(j, _):
            for u in range(8):
                hist_v[pl.ds(j * 128 + u * 16, 16)] = zeros16
            return 0

        lax.fori_loop(0, (_SC_L * _NB) // 128, _zero, 0)

        def _chunk(c, _):
            pltpu.sync_copy(data_hbm.at[pl.ds(base + c * _CHUNK, _CHUNK)],
                            buf_v)

            def _vec(i, _):
                key = buf_v[pl.ds(i * _SC_L, _SC_L)] & signmask
                keep = lax.shift_right_logical(key, fs) == fv
                bucket = lax.shift_right_logical(key, bs) & bm
                idx = lanes * _NB + bucket
                plsc.addupdate_scatter(hist_v, [idx], ones, mask=keep)
                return 0

            lax.fori_loop(0, _CHUNK // _SC_L, _vec, 0)
            return 0

        lax.fori_loop(0, _PER_W // _CHUNK, _chunk, 0)

        def _reduce(j, _):
            acc = zeros16
            for l in range(_SC_L):
                acc = acc + hist_v[pl.ds(l * _NB + j * 16, 16)]
            res_v[pl.ds(j * 16, 16)] = acc
            return 0

        lax.fori_loop(0, _NB // 16, _reduce, 0)
        pltpu.sync_copy(res_v, out_hbm.at[pl.ds(wid * _NB, _NB)])


def _splat(vals):
    return jnp.repeat(jnp.asarray(vals, jnp.int32), 16, total_repeat_length=16 * len(vals))


def _boundary_bucket(counts, rank):
    suffix = jnp.cumsum(counts[::-1])[::-1]
    b = jnp.max(jnp.where(suffix >= rank, jnp.arange(counts.shape[0]), 0))
    above = jnp.where(b + 1 < counts.shape[0], suffix[jnp.minimum(b + 1, counts.shape[0] - 1)], 0)
    return b, above


def _thresholds(k, v):
    kf = lax.bitcast_convert_type(k.reshape(-1), jnp.int32)
    vf = lax.bitcast_convert_type(v.reshape(-1), jnp.int32)
    par1 = _splat([31, 0, 19, _NB - 1] * 2)
    hk1, hv1 = _sc_hist(kf, vf, par1)
    ck1 = jnp.sum(hk1.reshape(_NW, _NB), axis=0)
    cv1 = jnp.sum(hv1.reshape(_NW, _NB), axis=0)
    bk1, above_k = _boundary_bucket(ck1, N_OUT)
    bv1, above_v = _boundary_bucket(cv1, N_OUT)

    par2 = jnp.concatenate([
        _splat([19]), jnp.full((16,), bk1, jnp.int32), _splat([8, 2047]),
        _splat([19]), jnp.full((16,), bv1, jnp.int32), _splat([8, 2047]),
    ])
    hk2, hv2 = _sc_hist(kf, vf, par2)
    ck2 = jnp.sum(hk2.reshape(_NW, _NB), axis=0)[:2048]
    cv2 = jnp.sum(hv2.reshape(_NW, _NB), axis=0)[:2048]
    bk2, _ = _boundary_bucket(ck2, N_OUT - above_k)
    bv2, _ = _boundary_bucket(cv2, N_OUT - above_v)

    t_k = lax.bitcast_convert_type(
        (bk1.astype(jnp.int32) << 19) | (bk2.astype(jnp.int32) << 8), jnp.float32)
    t_v = lax.bitcast_convert_type(
        (bv1.astype(jnp.int32) << 19) | (bv2.astype(jnp.int32) << 8), jnp.float32)
    return t_k, t_v


def _scale_kernel(k_ref, v_ref, thr_ref, kmax_ref, vmax_ref):
    h = pl.program_id(0)
    kabs = jnp.abs(k_ref[0, 0])
    vabs = jnp.abs(v_ref[0, 0])
    t_k = thr_ref[0:1, :]
    t_v = thr_ref[1:2, :]
    km = jnp.where(kabs < t_k, kabs, 0.0)
    rows = lax.broadcasted_iota(jnp.int32, (S, 1), 0)
    vmask = (vabs < t_v) & (rows >= SINK_LENGTH)
    vm = jnp.where(vmask, vabs, 0.0)
    kblk = jnp.max(km, axis=0, keepdims=True)
    vblk = jnp.max(vm, axis=0, keepdims=True)

    @pl.when(h == 0)
    def _():
        kmax_ref[...] = jnp.zeros_like(kmax_ref)
        vmax_ref[...] = jnp.zeros_like(vmax_ref)

    kmax_ref[...] = jnp.maximum(kmax_ref[...], kblk)
    vmax_ref[...] = jnp.maximum(vmax_ref[...], vblk)


def _masked_absmax(k, v, thr):
    out = pl.pallas_call(
        _scale_kernel,
        grid=(H,),
        in_specs=[
            pl.BlockSpec((1, 1, S, D), lambda h: (0, h, 0, 0)),
            pl.BlockSpec((1, 1, S, D), lambda h: (0, h, 0, 0)),
            pl.BlockSpec((8, D), lambda h: (0, 0)),
        ],
        out_specs=[
            pl.BlockSpec((1, D), lambda h: (0, 0)),
            pl.BlockSpec((1, D), lambda h: (0, 0)),
        ],
        out_shape=[
            jax.ShapeDtypeStruct((1, D), jnp.float32),
            jax.ShapeDtypeStruct((1, D), jnp.float32),
        ],
    )(k, v, thr)
    return out


def _flash_kernel(params_ref, q_ref, k_ref, v_ref, o_ref, krec_ref, vrec_ref):
    qb = pl.program_id(1)

    @pl.when(qb == 0)
    def _():
        kraw = k_ref[0, 0]
        vraw = v_ref[0, 0]
        ks = params_ref[0:1, :]
        vs = params_ref[1:2, :]
        t_k = params_ref[2:3, :]
        t_v = params_ref[3:4, :]
        kdq = jnp.clip(jnp.round(kraw / ks), -QMAX, QMAX) * ks
        krec_ref[...] = jnp.where(jnp.abs(kraw) >= t_k, kraw, kdq)
        vdq = jnp.clip(jnp.round(vraw / vs), -QMAX, QMAX) * vs
        rows = lax.broadcasted_iota(jnp.int32, (S, 1), 0)
        keep = (jnp.abs(vraw) >= t_v) | (rows < SINK_LENGTH)
        vrec_ref[...] = jnp.where(keep, vraw, vdq)

    qblk = q_ref[0, 0]
    s = lax.dot_general(
        qblk, krec_ref[...], (((1,), (1,)), ((), ())),
        preferred_element_type=jnp.float32,
        precision=lax.Precision.HIGHEST,
    ) / math.sqrt(float(D))
    m = jnp.max(s, axis=-1, keepdims=True)
    p = jnp.exp(s - m)
    denom = jnp.sum(p, axis=-1, keepdims=True)
    o = lax.dot_general(
        p, vrec_ref[...], (((1,), (0,)), ((), ())),
        preferred_element_type=jnp.float32,
        precision=lax.Precision.HIGHEST,
    )
    o_ref[0, 0] = o / denom


def _attention(params, q, k, v):
    return pl.pallas_call(
        _flash_kernel,
        grid=(H, S // BLK_Q),
        in_specs=[
            pl.BlockSpec((8, D), lambda h, qb: (0, 0)),
            pl.BlockSpec((1, 1, BLK_Q, D), lambda h, qb: (0, h, qb, 0)),
            pl.BlockSpec((1, 1, S, D), lambda h, qb: (0, h, 0, 0)),
            pl.BlockSpec((1, 1, S, D), lambda h, qb: (0, h, 0, 0)),
        ],
        out_specs=pl.BlockSpec((1, 1, BLK_Q, D), lambda h, qb: (0, h, qb, 0)),
        out_shape=jax.ShapeDtypeStruct((B, H, S, D), jnp.float32),
        scratch_shapes=[
            pltpu.VMEM((S, D), jnp.float32),
            pltpu.VMEM((S, D), jnp.float32),
        ],
    )(params, q, k, v)


def kernel(q_tensor, k_tensor, v_tensor):
    t_k, t_v = _thresholds(k_tensor, v_tensor)

    thr = jnp.zeros((8, D), jnp.float32)
    thr = thr.at[0, :].set(t_k)
    thr = thr.at[1, :].set(t_v)

    kmax, vmax = _masked_absmax(k_tensor, v_tensor, thr)
    k_scale = jnp.maximum(kmax[0], 1e-6) / QMAX
    v_scale = jnp.maximum(vmax[0], 1e-6) / QMAX

    params = jnp.zeros((8, D), jnp.float32)
    params = params.at[0, :].set(k_scale)
    params = params.at[1, :].set(v_scale)
    params = params.at[2, :].set(t_k)
    params = params.at[3, :].set(t_v)

    return _attention(params, q_tensor, k_tensor, v_tensor)


# speedup vs baseline: 18.2889x; 2.2725x over previous
import functools
import math

import jax
import jax.numpy as jnp
from jax import lax
from jax.experimental import pallas as pl
from jax.experimental.pallas import tpu as pltpu
from jax.experimental.pallas import tpu_sc as plsc

B, H, S, D = 1, 16, 2048, 128
N_ELEM = B * H * S * D
N_OUT = int(0.005 * N_ELEM)
SINK_LENGTH = 4
QMAX = 127.0
BLK_Q = 512


_SC_NC, _SC_NS, _SC_L = 2, 16, 16
_NW = _SC_NC * _SC_NS
_PER_W = N_ELEM // _NW
_CHUNK = 32768
_UNROLL = 8
_NB = 4096

_sc_mesh = plsc.VectorSubcoreMesh(core_axis_name="c", subcore_axis_name="s")


@functools.partial(
    pl.kernel,
    mesh=_sc_mesh,
    out_type=[
        jax.ShapeDtypeStruct((_NW * _NB,), jnp.int32),
        jax.ShapeDtypeStruct((_NW * _NB,), jnp.int32),
    ],
    scratch_types=[
        pltpu.VMEM((128,), jnp.int32),
        pltpu.VMEM((_CHUNK,), jnp.int32),
        pltpu.VMEM((_SC_L * _NB,), jnp.int32),
        pltpu.VMEM((_NB,), jnp.int32),
    ],
    compiler_params=pltpu.CompilerParams(needs_layout_passes=False),
)
def _sc_hist(k_hbm, v_hbm, par_hbm, outk_hbm, outv_hbm,
             par_v, buf_v, hist_v, res_v):
    wid = lax.axis_index("s") * _SC_NC + lax.axis_index("c")
    base = wid * _PER_W
    pltpu.sync_copy(par_hbm, par_v)
    lanes = lax.iota(jnp.int32, _SC_L)
    ones = jnp.ones((_SC_L,), jnp.int32)
    zeros16 = jnp.zeros((_SC_L,), jnp.int32)
    signmask = jnp.full((_SC_L,), 0x7FFFFFFF, jnp.int32)

    for t, (data_hbm, out_hbm) in enumerate(((k_hbm, outk_hbm),
                                             (v_hbm, outv_hbm))):
        fs = par_v[pl.ds((4 * t + 0) * 16, 16)]
        fv = par_v[pl.ds((4 * t + 1) * 16, 16)]
        bs = par_v[pl.ds((4 * t + 2) * 16, 16)]
        bm = par_v[pl.ds((4 * t + 3) * 16, 16)]

        def _zero(j, _):
            for u in range(8):
                hist_v[pl.ds(j * 128 + u * 16, 16)] = zeros16
            return 0

        lax.fori_loop(0, (_SC_L * _NB) // 128, _zero, 0)

        def _chunk(c, _):
            pltpu.sync_copy(data_hbm.at[pl.ds(base + c * _CHUNK, _CHUNK)],
                            buf_v)

            def _vec(i, _):
                for u in range(_UNROLL):
                    key = buf_v[pl.ds((i * _UNROLL + u) * _SC_L, _SC_L)] & signmask
                    keep = lax.shift_right_logical(key, fs) == fv
                    bucket = lax.shift_right_logical(key, bs) & bm
                    idx = lanes * _NB + bucket
                    plsc.addupdate_scatter(hist_v, [idx], ones, mask=keep)
                return 0

            lax.fori_loop(0, _CHUNK // (_SC_L * _UNROLL), _vec, 0)
            return 0

        lax.fori_loop(0, _PER_W // _CHUNK, _chunk, 0)

        def _reduce(j, _):
            acc = zeros16
            for l in range(_SC_L):
                acc = acc + hist_v[pl.ds(l * _NB + j * 16, 16)]
            res_v[pl.ds(j * 16, 16)] = acc
            return 0

        lax.fori_loop(0, _NB // 16, _reduce, 0)
        pltpu.sync_copy(res_v, out_hbm.at[pl.ds(wid * _NB, _NB)])


def _splat(vals):
    return jnp.repeat(jnp.asarray(vals, jnp.int32), 16, total_repeat_length=16 * len(vals))


def _boundary_bucket(counts, rank):
    suffix = jnp.cumsum(counts[::-1])[::-1]
    b = jnp.max(jnp.where(suffix >= rank, jnp.arange(counts.shape[0]), 0))
    above = jnp.where(b + 1 < counts.shape[0], suffix[jnp.minimum(b + 1, counts.shape[0] - 1)], 0)
    return b, above


def _thresholds(k, v):
    kf = lax.bitcast_convert_type(k.reshape(-1), jnp.int32)
    vf = lax.bitcast_convert_type(v.reshape(-1), jnp.int32)
    par1 = _splat([31, 0, 19, _NB - 1] * 2)
    hk1, hv1 = _sc_hist(kf, vf, par1)
    ck1 = jnp.sum(hk1.reshape(_NW, _NB), axis=0)
    cv1 = jnp.sum(hv1.reshape(_NW, _NB), axis=0)
    bk1, above_k = _boundary_bucket(ck1, N_OUT)
    bv1, above_v = _boundary_bucket(cv1, N_OUT)

    par2 = jnp.concatenate([
        _splat([19]), jnp.full((16,), bk1, jnp.int32), _splat([8, 2047]),
        _splat([19]), jnp.full((16,), bv1, jnp.int32), _splat([8, 2047]),
    ])
    hk2, hv2 = _sc_hist(kf, vf, par2)
    ck2 = jnp.sum(hk2.reshape(_NW, _NB), axis=0)[:2048]
    cv2 = jnp.sum(hv2.reshape(_NW, _NB), axis=0)[:2048]
    bk2, _ = _boundary_bucket(ck2, N_OUT - above_k)
    bv2, _ = _boundary_bucket(cv2, N_OUT - above_v)

    t_k = lax.bitcast_convert_type(
        (bk1.astype(jnp.int32) << 19) | (bk2.astype(jnp.int32) << 8), jnp.float32)
    t_v = lax.bitcast_convert_type(
        (bv1.astype(jnp.int32) << 19) | (bv2.astype(jnp.int32) << 8), jnp.float32)
    return t_k, t_v


def _scale_kernel(k_ref, v_ref, thr_ref, kmax_ref, vmax_ref):
    h = pl.program_id(0)
    kabs = jnp.abs(k_ref[0, 0])
    vabs = jnp.abs(v_ref[0, 0])
    t_k = thr_ref[0:1, :]
    t_v = thr_ref[1:2, :]
    km = jnp.where(kabs < t_k, kabs, 0.0)
    rows = lax.broadcasted_iota(jnp.int32, (S, 1), 0)
    vmask = (vabs < t_v) & (rows >= SINK_LENGTH)
    vm = jnp.where(vmask, vabs, 0.0)
    kblk = jnp.max(km, axis=0, keepdims=True)
    vblk = jnp.max(vm, axis=0, keepdims=True)

    @pl.when(h == 0)
    def _():
        kmax_ref[...] = jnp.zeros_like(kmax_ref)
        vmax_ref[...] = jnp.zeros_like(vmax_ref)

    kmax_ref[...] = jnp.maximum(kmax_ref[...], kblk)
    vmax_ref[...] = jnp.maximum(vmax_ref[...], vblk)


def _masked_absmax(k, v, thr):
    out = pl.pallas_call(
        _scale_kernel,
        grid=(H,),
        in_specs=[
            pl.BlockSpec((1, 1, S, D), lambda h: (0, h, 0, 0)),
            pl.BlockSpec((1, 1, S, D), lambda h: (0, h, 0, 0)),
            pl.BlockSpec((8, D), lambda h: (0, 0)),
        ],
        out_specs=[
            pl.BlockSpec((1, D), lambda h: (0, 0)),
            pl.BlockSpec((1, D), lambda h: (0, 0)),
        ],
        out_shape=[
            jax.ShapeDtypeStruct((1, D), jnp.float32),
            jax.ShapeDtypeStruct((1, D), jnp.float32),
        ],
    )(k, v, thr)
    return out


def _flash_kernel(params_ref, q_ref, k_ref, v_ref, o_ref, krec_ref, vrec_ref):
    qb = pl.program_id(1)

    @pl.when(qb == 0)
    def _():
        kraw = k_ref[0, 0]
        vraw = v_ref[0, 0]
        ks = params_ref[0:1, :]
        vs = params_ref[1:2, :]
        t_k = params_ref[2:3, :]
        t_v = params_ref[3:4, :]
        kdq = jnp.clip(jnp.round(kraw / ks), -QMAX, QMAX) * ks
        krec_ref[...] = jnp.where(jnp.abs(kraw) >= t_k, kraw, kdq)
        vdq = jnp.clip(jnp.round(vraw / vs), -QMAX, QMAX) * vs
        rows = lax.broadcasted_iota(jnp.int32, (S, 1), 0)
        keep = (jnp.abs(vraw) >= t_v) | (rows < SINK_LENGTH)
        vrec_ref[...] = jnp.where(keep, vraw, vdq)

    qblk = q_ref[0, 0] * (1.0 / math.sqrt(float(D)))
    s = lax.dot_general(
        qblk, krec_ref[...], (((1,), (1,)), ((), ())),
        preferred_element_type=jnp.float32,
        precision=lax.Precision.DEFAULT,
    )
    m = jnp.max(s, axis=-1, keepdims=True)
    p = jnp.exp(s - m)
    denom = jnp.sum(p, axis=-1, keepdims=True)
    o = lax.dot_general(
        p, vrec_ref[...], (((1,), (0,)), ((), ())),
        preferred_element_type=jnp.float32,
        precision=lax.Precision.DEFAULT,
    )
    o_ref[0, 0] = o / denom


def _attention(params, q, k, v):
    return pl.pallas_call(
        _flash_kernel,
        grid=(H, S // BLK_Q),
        in_specs=[
            pl.BlockSpec((8, D), lambda h, qb: (0, 0)),
            pl.BlockSpec((1, 1, BLK_Q, D), lambda h, qb: (0, h, qb, 0)),
            pl.BlockSpec((1, 1, S, D), lambda h, qb: (0, h, 0, 0)),
            pl.BlockSpec((1, 1, S, D), lambda h, qb: (0, h, 0, 0)),
        ],
        out_specs=pl.BlockSpec((1, 1, BLK_Q, D), lambda h, qb: (0, h, qb, 0)),
        out_shape=jax.ShapeDtypeStruct((B, H, S, D), jnp.float32),
        scratch_shapes=[
            pltpu.VMEM((S, D), jnp.float32),
            pltpu.VMEM((S, D), jnp.float32),
        ],
    )(params, q, k, v)


def kernel(q_tensor, k_tensor, v_tensor):
    t_k, t_v = _thresholds(k_tensor, v_tensor)

    thr = jnp.zeros((8, D), jnp.float32)
    thr = thr.at[0, :].set(t_k)
    thr = thr.at[1, :].set(t_v)

    kmax, vmax = _masked_absmax(k_tensor, v_tensor, thr)
    k_scale = jnp.maximum(kmax[0], 1e-6) / QMAX
    v_scale = jnp.maximum(vmax[0], 1e-6) / QMAX

    params = jnp.zeros((8, D), jnp.float32)
    params = params.at[0, :].set(k_scale)
    params = params.at[1, :].set(v_scale)
    params = params.at[2, :].set(t_k)
    params = params.at[3, :].set(t_v)

    return _attention(params, q_tensor, k_tensor, v_tensor)


# speedup vs baseline: 30.4309x; 1.6639x over previous
import functools
import math

import jax
import jax.numpy as jnp
from jax import lax
from jax.experimental import pallas as pl
from jax.experimental.pallas import tpu as pltpu
from jax.experimental.pallas import tpu_sc as plsc

B, H, S, D = 1, 16, 2048, 128
N_ELEM = B * H * S * D
N_OUT = int(0.005 * N_ELEM)
SINK_LENGTH = 4
QMAX = 127.0
BLK_Q = 512


_SC_NC, _SC_NS, _SC_L = 2, 16, 16
_NW = _SC_NC * _SC_NS
_PER_W = N_ELEM // _NW
_CHUNK = 32768
_UNROLL = 8
_NB = 4096

_sc_mesh = plsc.VectorSubcoreMesh(core_axis_name="c", subcore_axis_name="s")


@functools.partial(
    pl.kernel,
    mesh=_sc_mesh,
    out_type=[
        jax.ShapeDtypeStruct((_NW * _NB,), jnp.int32),
        jax.ShapeDtypeStruct((_NW * _NB,), jnp.int32),
    ],
    scratch_types=[
        pltpu.VMEM((128,), jnp.int32),
        pltpu.VMEM((_CHUNK,), jnp.int32),
        pltpu.VMEM((_SC_L * _NB,), jnp.int32),
        pltpu.VMEM((_NB,), jnp.int32),
    ],
    compiler_params=pltpu.CompilerParams(needs_layout_passes=False),
)
def _sc_hist(k_hbm, v_hbm, par_hbm, outk_hbm, outv_hbm,
             par_v, buf_v, hist_v, res_v):
    wid = lax.axis_index("s") * _SC_NC + lax.axis_index("c")
    base = wid * _PER_W
    pltpu.sync_copy(par_hbm, par_v)
    lanes = lax.iota(jnp.int32, _SC_L)
    ones = jnp.ones((_SC_L,), jnp.int32)
    zeros16 = jnp.zeros((_SC_L,), jnp.int32)
    signmask = jnp.full((_SC_L,), 0x7FFFFFFF, jnp.int32)

    for t, (data_hbm, out_hbm) in enumerate(((k_hbm, outk_hbm),
                                             (v_hbm, outv_hbm))):
        fs = par_v[pl.ds((4 * t + 0) * 16, 16)]
        fv = par_v[pl.ds((4 * t + 1) * 16, 16)]
        bs = par_v[pl.ds((4 * t + 2) * 16, 16)]
        bm = par_v[pl.ds((4 * t + 3) * 16, 16)]

        @plsc.parallel_loop(0, (_SC_L * _NB) // 16, unroll=8)
        def _zero(j):
            hist_v[pl.ds(j * 16, 16)] = zeros16

        def _chunk(c, _):
            pltpu.sync_copy(data_hbm.at[pl.ds(base + c * _CHUNK, _CHUNK)],
                            buf_v)

            @plsc.parallel_loop(0, _CHUNK // _SC_L, unroll=_UNROLL)
            def _vec(i):
                key = buf_v[pl.ds(i * _SC_L, _SC_L)] & signmask
                keep = lax.shift_right_logical(key, fs) == fv
                bucket = lax.shift_right_logical(key, bs) & bm
                idx = lanes * _NB + bucket
                plsc.addupdate_scatter(hist_v, [idx], ones, mask=keep)

            return 0

        lax.fori_loop(0, _PER_W // _CHUNK, _chunk, 0)

        @plsc.parallel_loop(0, _NB // 16, unroll=2)
        def _reduce(j):
            acc = hist_v[pl.ds(j * 16, 16)]
            for l in range(1, _SC_L):
                acc = acc + hist_v[pl.ds(l * _NB + j * 16, 16)]
            res_v[pl.ds(j * 16, 16)] = acc
        pltpu.sync_copy(res_v, out_hbm.at[pl.ds(wid * _NB, _NB)])


def _splat(vals):
    return jnp.repeat(jnp.asarray(vals, jnp.int32), 16, total_repeat_length=16 * len(vals))


def _boundary_bucket(counts, rank):
    suffix = jnp.cumsum(counts[::-1])[::-1]
    b = jnp.max(jnp.where(suffix >= rank, jnp.arange(counts.shape[0]), 0))
    above = jnp.where(b + 1 < counts.shape[0], suffix[jnp.minimum(b + 1, counts.shape[0] - 1)], 0)
    return b, above


def _thresholds(k, v):
    kf = lax.bitcast_convert_type(k.reshape(-1), jnp.int32)
    vf = lax.bitcast_convert_type(v.reshape(-1), jnp.int32)
    par1 = _splat([31, 0, 19, _NB - 1] * 2)
    hk1, hv1 = _sc_hist(kf, vf, par1)
    ck1 = jnp.sum(hk1.reshape(_NW, _NB), axis=0)
    cv1 = jnp.sum(hv1.reshape(_NW, _NB), axis=0)
    bk1, above_k = _boundary_bucket(ck1, N_OUT)
    bv1, above_v = _boundary_bucket(cv1, N_OUT)

    par2 = jnp.concatenate([
        _splat([19]), jnp.full((16,), bk1, jnp.int32), _splat([8, 2047]),
        _splat([19]), jnp.full((16,), bv1, jnp.int32), _splat([8, 2047]),
    ])
    hk2, hv2 = _sc_hist(kf, vf, par2)
    ck2 = jnp.sum(hk2.reshape(_NW, _NB), axis=0)[:2048]
    cv2 = jnp.sum(hv2.reshape(_NW, _NB), axis=0)[:2048]
    bk2, _ = _boundary_bucket(ck2, N_OUT - above_k)
    bv2, _ = _boundary_bucket(cv2, N_OUT - above_v)

    t_k = lax.bitcast_convert_type(
        (bk1.astype(jnp.int32) << 19) | (bk2.astype(jnp.int32) << 8), jnp.float32)
    t_v = lax.bitcast_convert_type(
        (bv1.astype(jnp.int32) << 19) | (bv2.astype(jnp.int32) << 8), jnp.float32)
    return t_k, t_v


def _scale_kernel(k_ref, v_ref, thr_ref, kmax_ref, vmax_ref):
    h = pl.program_id(0)
    kabs = jnp.abs(k_ref[0, 0])
    vabs = jnp.abs(v_ref[0, 0])
    t_k = thr_ref[0:1, :]
    t_v = thr_ref[1:2, :]
    km = jnp.where(kabs < t_k, kabs, 0.0)
    rows = lax.broadcasted_iota(jnp.int32, (S, 1), 0)
    vmask = (vabs < t_v) & (rows >= SINK_LENGTH)
    vm = jnp.where(vmask, vabs, 0.0)
    kblk = jnp.max(km, axis=0, keepdims=True)
    vblk = jnp.max(vm, axis=0, keepdims=True)

    @pl.when(h == 0)
    def _():
        kmax_ref[...] = jnp.zeros_like(kmax_ref)
        vmax_ref[...] = jnp.zeros_like(vmax_ref)

    kmax_ref[...] = jnp.maximum(kmax_ref[...], kblk)
    vmax_ref[...] = jnp.maximum(vmax_ref[...], vblk)


def _masked_absmax(k, v, thr):
    out = pl.pallas_call(
        _scale_kernel,
        grid=(H,),
        in_specs=[
            pl.BlockSpec((1, 1, S, D), lambda h: (0, h, 0, 0)),
            pl.BlockSpec((1, 1, S, D), lambda h: (0, h, 0, 0)),
            pl.BlockSpec((8, D), lambda h: (0, 0)),
        ],
        out_specs=[
            pl.BlockSpec((1, D), lambda h: (0, 0)),
            pl.BlockSpec((1, D), lambda h: (0, 0)),
        ],
        out_shape=[
            jax.ShapeDtypeStruct((1, D), jnp.float32),
            jax.ShapeDtypeStruct((1, D), jnp.float32),
        ],
    )(k, v, thr)
    return out


def _flash_kernel(params_ref, q_ref, k_ref, v_ref, o_ref, krec_ref, vrec_ref):
    qb = pl.program_id(1)

    @pl.when(qb == 0)
    def _():
        kraw = k_ref[0, 0]
        vraw = v_ref[0, 0]
        ks = params_ref[0:1, :]
        vs = params_ref[1:2, :]
        t_k = params_ref[2:3, :]
        t_v = params_ref[3:4, :]
        kdq = jnp.clip(jnp.round(kraw / ks), -QMAX, QMAX) * ks
        krec_ref[...] = jnp.where(jnp.abs(kraw) >= t_k, kraw, kdq)
        vdq = jnp.clip(jnp.round(vraw / vs), -QMAX, QMAX) * vs
        rows = lax.broadcasted_iota(jnp.int32, (S, 1), 0)
        keep = (jnp.abs(vraw) >= t_v) | (rows < SINK_LENGTH)
        vrec_ref[...] = jnp.where(keep, vraw, vdq)

    qblk = q_ref[0, 0] * (1.0 / math.sqrt(float(D)))
    s = lax.dot_general(
        qblk, krec_ref[...], (((1,), (1,)), ((), ())),
        preferred_element_type=jnp.float32,
        precision=lax.Precision.DEFAULT,
    )
    m = jnp.max(s, axis=-1, keepdims=True)
    p = jnp.exp(s - m)
    denom = jnp.sum(p, axis=-1, keepdims=True)
    o = lax.dot_general(
        p, vrec_ref[...], (((1,), (0,)), ((), ())),
        preferred_element_type=jnp.float32,
        precision=lax.Precision.DEFAULT,
    )
    o_ref[0, 0] = o / denom


def _attention(params, q, k, v):
    return pl.pallas_call(
        _flash_kernel,
        grid=(H, S // BLK_Q),
        in_specs=[
            pl.BlockSpec((8, D), lambda h, qb: (0, 0)),
            pl.BlockSpec((1, 1, BLK_Q, D), lambda h, qb: (0, h, qb, 0)),
            pl.BlockSpec((1, 1, S, D), lambda h, qb: (0, h, 0, 0)),
            pl.BlockSpec((1, 1, S, D), lambda h, qb: (0, h, 0, 0)),
        ],
        out_specs=pl.BlockSpec((1, 1, BLK_Q, D), lambda h, qb: (0, h, qb, 0)),
        out_shape=jax.ShapeDtypeStruct((B, H, S, D), jnp.float32),
        scratch_shapes=[
            pltpu.VMEM((S, D), jnp.float32),
            pltpu.VMEM((S, D), jnp.float32),
        ],
    )(params, q, k, v)


def kernel(q_tensor, k_tensor, v_tensor):
    t_k, t_v = _thresholds(k_tensor, v_tensor)

    thr = jnp.zeros((8, D), jnp.float32)
    thr = thr.at[0, :].set(t_k)
    thr = thr.at[1, :].set(t_v)

    kmax, vmax = _masked_absmax(k_tensor, v_tensor, thr)
    k_scale = jnp.maximum(kmax[0], 1e-6) / QMAX
    v_scale = jnp.maximum(vmax[0], 1e-6) / QMAX

    params = jnp.zeros((8, D), jnp.float32)
    params = params.at[0, :].set(k_scale)
    params = params.at[1, :].set(v_scale)
    params = params.at[2, :].set(t_k)
    params = params.at[3, :].set(t_v)

    return _attention(params, q_tensor, k_tensor, v_tensor)


# speedup vs baseline: 36.4422x; 1.1975x over previous
import functools
import math

import jax
import jax.numpy as jnp
from jax import lax
from jax.experimental import pallas as pl
from jax.experimental.pallas import tpu as pltpu
from jax.experimental.pallas import tpu_sc as plsc

B, H, S, D = 1, 16, 2048, 128
N_ELEM = B * H * S * D
N_OUT = int(0.005 * N_ELEM)
SINK_LENGTH = 4
QMAX = 127.0
BLK_Q = 512


_SC_NC, _SC_NS, _SC_L = 2, 16, 16
_NW = _SC_NC * _SC_NS
_PER_W = N_ELEM // _NW
_CHUNK = 32768
_UNROLL = 8
_NB = 4096

_sc_mesh = plsc.VectorSubcoreMesh(core_axis_name="c", subcore_axis_name="s")


@functools.partial(
    pl.kernel,
    mesh=_sc_mesh,
    out_type=[
        jax.ShapeDtypeStruct((_NW * _NB,), jnp.int32),
        jax.ShapeDtypeStruct((_NW * _NB,), jnp.int32),
    ],
    scratch_types=[
        pltpu.VMEM((128,), jnp.int32),
        pltpu.VMEM((_CHUNK,), jnp.int32),
        pltpu.VMEM((_SC_L * _NB,), jnp.int32),
        pltpu.VMEM((_NB,), jnp.int32),
    ],
    compiler_params=pltpu.CompilerParams(needs_layout_passes=False),
)
def _sc_hist(k_hbm, v_hbm, par_hbm, outk_hbm, outv_hbm,
             par_v, buf_v, hist_v, res_v):
    wid = lax.axis_index("s") * _SC_NC + lax.axis_index("c")
    base = wid * _PER_W
    pltpu.sync_copy(par_hbm, par_v)
    lanes = lax.iota(jnp.int32, _SC_L)
    ones = jnp.ones((_SC_L,), jnp.int32)
    zeros16 = jnp.zeros((_SC_L,), jnp.int32)
    signmask = jnp.full((_SC_L,), 0x7FFFFFFF, jnp.int32)

    for t, (data_hbm, out_hbm) in enumerate(((k_hbm, outk_hbm),
                                             (v_hbm, outv_hbm))):
        fs = par_v[pl.ds((4 * t + 0) * 16, 16)]
        fv = par_v[pl.ds((4 * t + 1) * 16, 16)]
        bs = par_v[pl.ds((4 * t + 2) * 16, 16)]
        bm = par_v[pl.ds((4 * t + 3) * 16, 16)]

        @plsc.parallel_loop(0, (_SC_L * _NB) // 16, unroll=8)
        def _zero(j):
            hist_v[pl.ds(j * 16, 16)] = zeros16

        def _chunk(c, _):
            pltpu.sync_copy(data_hbm.at[pl.ds(base + c * _CHUNK, _CHUNK)],
                            buf_v)

            @plsc.parallel_loop(0, _CHUNK // _SC_L, unroll=_UNROLL)
            def _vec(i):
                key = buf_v[pl.ds(i * _SC_L, _SC_L)] & signmask
                keep = lax.shift_right_logical(key, fs) == fv
                bucket = lax.shift_right_logical(key, bs) & bm
                idx = lanes * _NB + bucket
                plsc.addupdate_scatter(hist_v, [idx], ones, mask=keep)

            return 0

        lax.fori_loop(0, _PER_W // _CHUNK, _chunk, 0)

        @plsc.parallel_loop(0, _NB // 16, unroll=2)
        def _reduce(j):
            acc = hist_v[pl.ds(j * 16, 16)]
            for l in range(1, _SC_L):
                acc = acc + hist_v[pl.ds(l * _NB + j * 16, 16)]
            res_v[pl.ds(j * 16, 16)] = acc
        pltpu.sync_copy(res_v, out_hbm.at[pl.ds(wid * _NB, _NB)])


def _splat(vals):
    return jnp.repeat(jnp.asarray(vals, jnp.int32), 16, total_repeat_length=16 * len(vals))


def _boundary_bucket(counts, rank):
    suffix = jnp.cumsum(counts[::-1])[::-1]
    b = jnp.max(jnp.where(suffix >= rank, jnp.arange(counts.shape[0]), 0))
    above = jnp.where(b + 1 < counts.shape[0], suffix[jnp.minimum(b + 1, counts.shape[0] - 1)], 0)
    return b, above


def _thresholds(k, v):
    kf = lax.bitcast_convert_type(k.reshape(-1), jnp.int32)
    vf = lax.bitcast_convert_type(v.reshape(-1), jnp.int32)
    par1 = _splat([31, 0, 19, _NB - 1] * 2)
    hk1, hv1 = _sc_hist(kf, vf, par1)
    ck1 = jnp.sum(hk1.reshape(_NW, _NB), axis=0)
    cv1 = jnp.sum(hv1.reshape(_NW, _NB), axis=0)
    bk1, above_k = _boundary_bucket(ck1, N_OUT)
    bv1, above_v = _boundary_bucket(cv1, N_OUT)

    par2 = jnp.concatenate([
        _splat([19]), jnp.full((16,), bk1, jnp.int32), _splat([8, 2047]),
        _splat([19]), jnp.full((16,), bv1, jnp.int32), _splat([8, 2047]),
    ])
    hk2, hv2 = _sc_hist(kf, vf, par2)
    ck2 = jnp.sum(hk2.reshape(_NW, _NB), axis=0)[:2048]
    cv2 = jnp.sum(hv2.reshape(_NW, _NB), axis=0)[:2048]
    bk2, _ = _boundary_bucket(ck2, N_OUT - above_k)
    bv2, _ = _boundary_bucket(cv2, N_OUT - above_v)

    t_k = lax.bitcast_convert_type(
        (bk1.astype(jnp.int32) << 19) | (bk2.astype(jnp.int32) << 8), jnp.float32)
    t_v = lax.bitcast_convert_type(
        (bv1.astype(jnp.int32) << 19) | (bv2.astype(jnp.int32) << 8), jnp.float32)
    return t_k, t_v


def _scale_kernel(k_ref, v_ref, thr_ref, kmax_ref, vmax_ref):
    h = pl.program_id(0)
    kabs = jnp.abs(k_ref[0, 0])
    vabs = jnp.abs(v_ref[0, 0])
    t_k = thr_ref[0:1, :]
    t_v = thr_ref[1:2, :]
    km = jnp.where(kabs < t_k, kabs, 0.0)
    rows = lax.broadcasted_iota(jnp.int32, (S, 1), 0)
    vmask = (vabs < t_v) & (rows >= SINK_LENGTH)
    vm = jnp.where(vmask, vabs, 0.0)
    kblk = jnp.max(km, axis=0, keepdims=True)
    vblk = jnp.max(vm, axis=0, keepdims=True)

    @pl.when(h == 0)
    def _():
        kmax_ref[...] = jnp.zeros_like(kmax_ref)
        vmax_ref[...] = jnp.zeros_like(vmax_ref)

    kmax_ref[...] = jnp.maximum(kmax_ref[...], kblk)
    vmax_ref[...] = jnp.maximum(vmax_ref[...], vblk)


def _masked_absmax(k, v, thr):
    out = pl.pallas_call(
        _scale_kernel,
        grid=(H,),
        in_specs=[
            pl.BlockSpec((1, 1, S, D), lambda h: (0, h, 0, 0)),
            pl.BlockSpec((1, 1, S, D), lambda h: (0, h, 0, 0)),
            pl.BlockSpec((8, D), lambda h: (0, 0)),
        ],
        out_specs=[
            pl.BlockSpec((1, D), lambda h: (0, 0)),
            pl.BlockSpec((1, D), lambda h: (0, 0)),
        ],
        out_shape=[
            jax.ShapeDtypeStruct((1, D), jnp.float32),
            jax.ShapeDtypeStruct((1, D), jnp.float32),
        ],
    )(k, v, thr)
    return out


def _flash_kernel(params_ref, q_ref, k_ref, v_ref, o_ref, krec_ref, vrec_ref):
    qb = pl.program_id(1)

    @pl.when(qb == 0)
    def _():
        kraw = k_ref[0, 0]
        vraw = v_ref[0, 0]
        ks = params_ref[0:1, :]
        vs = params_ref[1:2, :]
        t_k = params_ref[2:3, :]
        t_v = params_ref[3:4, :]
        kdq = jnp.clip(jnp.round(kraw / ks), -QMAX, QMAX) * ks
        krec_ref[...] = jnp.where(jnp.abs(kraw) >= t_k, kraw, kdq)
        vdq = jnp.clip(jnp.round(vraw / vs), -QMAX, QMAX) * vs
        rows = lax.broadcasted_iota(jnp.int32, (S, 1), 0)
        keep = (jnp.abs(vraw) >= t_v) | (rows < SINK_LENGTH)
        vrec_ref[...] = jnp.where(keep, vraw, vdq)

    qblk = q_ref[0, 0] * (1.0 / math.sqrt(float(D)))
    n_chunks = 4
    half = S // n_chunks
    os_, ms_, ls_ = [], [], []
    for c in range(n_chunks):
        s = lax.dot_general(
            qblk, krec_ref[pl.ds(c * half, half), :], (((1,), (1,)), ((), ())),
            preferred_element_type=jnp.float32,
            precision=lax.Precision.DEFAULT,
        )
        m = jnp.max(s, axis=-1, keepdims=True)
        p = jnp.exp(s - m)
        l = jnp.sum(p, axis=-1, keepdims=True)
        o = lax.dot_general(
            p, vrec_ref[pl.ds(c * half, half), :], (((1,), (0,)), ((), ())),
            preferred_element_type=jnp.float32,
            precision=lax.Precision.DEFAULT,
        )
        os_.append(o); ms_.append(m); ls_.append(l)
    mm = functools.reduce(jnp.maximum, ms_)
    onum = None
    oden = None
    for c in range(n_chunks):
        a = jnp.exp(ms_[c] - mm)
        onum = os_[c] * a if onum is None else onum + os_[c] * a
        oden = ls_[c] * a if oden is None else oden + ls_[c] * a
    o_ref[0, 0] = onum / oden


def _attention(params, q, k, v):
    return pl.pallas_call(
        _flash_kernel,
        grid=(H, S // BLK_Q),
        in_specs=[
            pl.BlockSpec((8, D), lambda h, qb: (0, 0)),
            pl.BlockSpec((1, 1, BLK_Q, D), lambda h, qb: (0, h, qb, 0)),
            pl.BlockSpec((1, 1, S, D), lambda h, qb: (0, h, 0, 0)),
            pl.BlockSpec((1, 1, S, D), lambda h, qb: (0, h, 0, 0)),
        ],
        out_specs=pl.BlockSpec((1, 1, BLK_Q, D), lambda h, qb: (0, h, qb, 0)),
        out_shape=jax.ShapeDtypeStruct((B, H, S, D), jnp.float32),
        scratch_shapes=[
            pltpu.VMEM((S, D), jnp.float32),
            pltpu.VMEM((S, D), jnp.float32),
        ],
    )(params, q, k, v)


def kernel(q_tensor, k_tensor, v_tensor):
    t_k, t_v = _thresholds(k_tensor, v_tensor)

    thr = jnp.zeros((8, D), jnp.float32)
    thr = thr.at[0, :].set(t_k)
    thr = thr.at[1, :].set(t_v)

    kmax, vmax = _masked_absmax(k_tensor, v_tensor, thr)
    k_scale = jnp.maximum(kmax[0], 1e-6) / QMAX
    v_scale = jnp.maximum(vmax[0], 1e-6) / QMAX

    params = jnp.zeros((8, D), jnp.float32)
    params = params.at[0, :].set(k_scale)
    params = params.at[1, :].set(v_scale)
    params = params.at[2, :].set(t_k)
    params = params.at[3, :].set(t_v)

    return _attention(params, q_tensor, k_tensor, v_tensor)


# speedup vs baseline: 38.1453x; 1.0467x over previous
import functools
import math

import jax
import jax.numpy as jnp
from jax import lax
from jax.experimental import pallas as pl
from jax.experimental.pallas import tpu as pltpu
from jax.experimental.pallas import tpu_sc as plsc

B, H, S, D = 1, 16, 2048, 128
N_ELEM = B * H * S * D
N_OUT = int(0.005 * N_ELEM)
SINK_LENGTH = 4
QMAX = 127.0
BLK_Q = 512


_SC_NC, _SC_NS, _SC_L = 2, 16, 16
_NW = _SC_NC * _SC_NS
_PER_W = N_ELEM // _NW
_CHUNK = 16384
_UNROLL = 8
_NB = 4096

_sc_mesh = plsc.VectorSubcoreMesh(core_axis_name="c", subcore_axis_name="s")


@functools.partial(
    pl.kernel,
    mesh=_sc_mesh,
    out_type=[
        jax.ShapeDtypeStruct((_NW * _NB,), jnp.int32),
        jax.ShapeDtypeStruct((_NW * _NB,), jnp.int32),
    ],
    scratch_types=[
        pltpu.VMEM((128,), jnp.int32),
        pltpu.VMEM((_CHUNK,), jnp.int32),
        pltpu.VMEM((_CHUNK,), jnp.int32),
        pltpu.VMEM((_SC_L * _NB,), jnp.int32),
        pltpu.VMEM((_NB,), jnp.int32),
        pltpu.SemaphoreType.DMA,
        pltpu.SemaphoreType.DMA,
    ],
    compiler_params=pltpu.CompilerParams(needs_layout_passes=False),
)
def _sc_hist(k_hbm, v_hbm, par_hbm, outk_hbm, outv_hbm,
             par_v, buf_a, buf_b, hist_v, res_v, sem_a, sem_b):
    wid = lax.axis_index("s") * _SC_NC + lax.axis_index("c")
    base = wid * _PER_W
    pltpu.sync_copy(par_hbm, par_v)
    lanes = lax.iota(jnp.int32, _SC_L)
    ones = jnp.ones((_SC_L,), jnp.int32)
    zeros16 = jnp.zeros((_SC_L,), jnp.int32)
    signmask = jnp.full((_SC_L,), 0x7FFFFFFF, jnp.int32)

    for t, (data_hbm, out_hbm) in enumerate(((k_hbm, outk_hbm),
                                             (v_hbm, outv_hbm))):
        fs = par_v[pl.ds((4 * t + 0) * 16, 16)]
        fv = par_v[pl.ds((4 * t + 1) * 16, 16)]
        bs = par_v[pl.ds((4 * t + 2) * 16, 16)]
        bm = par_v[pl.ds((4 * t + 3) * 16, 16)]

        @plsc.parallel_loop(0, (_SC_L * _NB) // 16, unroll=8)
        def _zero(j):
            hist_v[pl.ds(j * 16, 16)] = zeros16

        n_chunks = _PER_W // _CHUNK
        bufs = (buf_a, buf_b)
        sems = (sem_a, sem_b)
        handles = [pltpu.async_copy(
            data_hbm.at[pl.ds(base, _CHUNK)], buf_a, sem_a)]
        for c in range(n_chunks):
            if c + 1 < n_chunks:
                handles.append(pltpu.async_copy(
                    data_hbm.at[pl.ds(base + (c + 1) * _CHUNK, _CHUNK)],
                    bufs[(c + 1) % 2], sems[(c + 1) % 2]))
            handles[c].wait()
            buf_v = bufs[c % 2]

            @plsc.parallel_loop(0, _CHUNK // _SC_L, unroll=_UNROLL)
            def _vec(i, buf_v=buf_v):
                key = buf_v[pl.ds(i * _SC_L, _SC_L)] & signmask
                keep = lax.shift_right_logical(key, fs) == fv
                bucket = lax.shift_right_logical(key, bs) & bm
                idx = lanes * _NB + bucket
                plsc.addupdate_scatter(hist_v, [idx], ones, mask=keep)

        @plsc.parallel_loop(0, _NB // 16, unroll=2)
        def _reduce(j):
            acc = hist_v[pl.ds(j * 16, 16)]
            for l in range(1, _SC_L):
                acc = acc + hist_v[pl.ds(l * _NB + j * 16, 16)]
            res_v[pl.ds(j * 16, 16)] = acc
        pltpu.sync_copy(res_v, out_hbm.at[pl.ds(wid * _NB, _NB)])


def _splat(vals):
    return jnp.repeat(jnp.asarray(vals, jnp.int32), 16, total_repeat_length=16 * len(vals))


def _boundary2(hk, hv, ranks, nb):
    c = jnp.stack([hk, hv]).reshape(2, _NW, _NB).sum(axis=1)[:, :nb]
    suffix = jnp.cumsum(c[:, ::-1], axis=1)[:, ::-1]
    iota = jnp.arange(nb)[None, :]
    b = jnp.max(jnp.where(suffix >= ranks[:, None], iota, 0), axis=1)
    shifted = jnp.concatenate(
        [suffix[:, 1:], jnp.zeros((2, 1), suffix.dtype)], axis=1)
    above = jnp.take_along_axis(shifted, b[:, None], axis=1)[:, 0]
    return b.astype(jnp.int32), above.astype(jnp.int32)


def _thresholds(k, v):
    kf = lax.bitcast_convert_type(k.reshape(-1), jnp.int32)
    vf = lax.bitcast_convert_type(v.reshape(-1), jnp.int32)
    par1 = _splat([31, 0, 19, _NB - 1] * 2)
    hk1, hv1 = _sc_hist(kf, vf, par1)
    b1, above = _boundary2(hk1, hv1, jnp.full((2,), N_OUT), _NB)

    par2 = jnp.concatenate([
        _splat([19]), jnp.full((16,), b1[0], jnp.int32), _splat([8, 2047]),
        _splat([19]), jnp.full((16,), b1[1], jnp.int32), _splat([8, 2047]),
    ])
    hk2, hv2 = _sc_hist(kf, vf, par2)
    b2, _ = _boundary2(hk2, hv2, N_OUT - above, 2048)

    t = lax.bitcast_convert_type((b1 << 19) | (b2 << 8), jnp.float32)
    return t[0], t[1]


def _scale_kernel(k_ref, v_ref, thr_ref, kmax_ref, vmax_ref):
    h = pl.program_id(0)
    kabs = jnp.abs(k_ref[0, 0])
    vabs = jnp.abs(v_ref[0, 0])
    t_k = thr_ref[0:1, :]
    t_v = thr_ref[1:2, :]
    km = jnp.where(kabs < t_k, kabs, 0.0)
    rows = lax.broadcasted_iota(jnp.int32, (S, 1), 0)
    vmask = (vabs < t_v) & (rows >= SINK_LENGTH)
    vm = jnp.where(vmask, vabs, 0.0)
    kblk = jnp.max(km, axis=0, keepdims=True)
    vblk = jnp.max(vm, axis=0, keepdims=True)

    @pl.when(h == 0)
    def _():
        kmax_ref[...] = jnp.zeros_like(kmax_ref)
        vmax_ref[...] = jnp.zeros_like(vmax_ref)

    kmax_ref[...] = jnp.maximum(kmax_ref[...], kblk)
    vmax_ref[...] = jnp.maximum(vmax_ref[...], vblk)


def _masked_absmax(k, v, thr):
    out = pl.pallas_call(
        _scale_kernel,
        grid=(H,),
        in_specs=[
            pl.BlockSpec((1, 1, S, D), lambda h: (0, h, 0, 0)),
            pl.BlockSpec((1, 1, S, D), lambda h: (0, h, 0, 0)),
            pl.BlockSpec((8, D), lambda h: (0, 0)),
        ],
        out_specs=[
            pl.BlockSpec((1, D), lambda h: (0, 0)),
            pl.BlockSpec((1, D), lambda h: (0, 0)),
        ],
        out_shape=[
            jax.ShapeDtypeStruct((1, D), jnp.float32),
            jax.ShapeDtypeStruct((1, D), jnp.float32),
        ],
    )(k, v, thr)
    return out


def _flash_kernel(params_ref, q_ref, k_ref, v_ref, o_ref, krec_ref, vrec_ref):
    qb = pl.program_id(1)

    @pl.when(qb == 0)
    def _():
        kraw = k_ref[0, 0]
        vraw = v_ref[0, 0]
        ks = params_ref[0:1, :]
        vs = params_ref[1:2, :]
        t_k = params_ref[2:3, :]
        t_v = params_ref[3:4, :]
        kdq = jnp.clip(jnp.round(kraw / ks), -QMAX, QMAX) * ks
        krec_ref[...] = jnp.where(jnp.abs(kraw) >= t_k, kraw, kdq)
        vdq = jnp.clip(jnp.round(vraw / vs), -QMAX, QMAX) * vs
        rows = lax.broadcasted_iota(jnp.int32, (S, 1), 0)
        keep = (jnp.abs(vraw) >= t_v) | (rows < SINK_LENGTH)
        vrec_ref[...] = jnp.where(keep, vraw, vdq)

    qblk = q_ref[0, 0] * (1.0 / math.sqrt(float(D)))
    n_chunks = 4
    half = S // n_chunks
    os_, ms_, ls_ = [], [], []
    for c in range(n_chunks):
        s = lax.dot_general(
            qblk, krec_ref[pl.ds(c * half, half), :], (((1,), (1,)), ((), ())),
            preferred_element_type=jnp.float32,
            precision=lax.Precision.DEFAULT,
        )
        m = jnp.max(s, axis=-1, keepdims=True)
        p = jnp.exp(s - m)
        l = jnp.sum(p, axis=-1, keepdims=True)
        o = lax.dot_general(
            p, vrec_ref[pl.ds(c * half, half), :], (((1,), (0,)), ((), ())),
            preferred_element_type=jnp.float32,
            precision=lax.Precision.DEFAULT,
        )
        os_.append(o); ms_.append(m); ls_.append(l)
    mm = functools.reduce(jnp.maximum, ms_)
    onum = None
    oden = None
    for c in range(n_chunks):
        a = jnp.exp(ms_[c] - mm)
        onum = os_[c] * a if onum is None else onum + os_[c] * a
        oden = ls_[c] * a if oden is None else oden + ls_[c] * a
    o_ref[0, 0] = onum / oden


def _attention(params, q, k, v):
    return pl.pallas_call(
        _flash_kernel,
        grid=(H, S // BLK_Q),
        in_specs=[
            pl.BlockSpec((8, D), lambda h, qb: (0, 0)),
            pl.BlockSpec((1, 1, BLK_Q, D), lambda h, qb: (0, h, qb, 0)),
            pl.BlockSpec((1, 1, S, D), lambda h, qb: (0, h, 0, 0)),
            pl.BlockSpec((1, 1, S, D), lambda h, qb: (0, h, 0, 0)),
        ],
        out_specs=pl.BlockSpec((1, 1, BLK_Q, D), lambda h, qb: (0, h, qb, 0)),
        out_shape=jax.ShapeDtypeStruct((B, H, S, D), jnp.float32),
        scratch_shapes=[
            pltpu.VMEM((S, D), jnp.float32),
            pltpu.VMEM((S, D), jnp.float32),
        ],
    )(params, q, k, v)


def kernel(q_tensor, k_tensor, v_tensor):
    t_k, t_v = _thresholds(k_tensor, v_tensor)

    thr = jnp.zeros((8, D), jnp.float32)
    thr = thr.at[0, :].set(t_k)
    thr = thr.at[1, :].set(t_v)

    kmax, vmax = _masked_absmax(k_tensor, v_tensor, thr)
    k_scale = jnp.maximum(kmax[0], 1e-6) / QMAX
    v_scale = jnp.maximum(vmax[0], 1e-6) / QMAX

    params = jnp.zeros((8, D), jnp.float32)
    params = params.at[0, :].set(k_scale)
    params = params.at[1, :].set(v_scale)
    params = params.at[2, :].set(t_k)
    params = params.at[3, :].set(t_v)

    return _attention(params, q_tensor, k_tensor, v_tensor)


# speedup vs baseline: 39.8943x; 1.0459x over previous
import functools
import math

import jax
import jax.numpy as jnp
from jax import lax
from jax.experimental import pallas as pl
from jax.experimental.pallas import tpu as pltpu
from jax.experimental.pallas import tpu_sc as plsc

B, H, S, D = 1, 16, 2048, 128
N_ELEM = B * H * S * D
N_OUT = int(0.005 * N_ELEM)
SINK_LENGTH = 4
QMAX = 127.0
BLK_Q = 512


_SC_NC, _SC_NS, _SC_L = 2, 16, 16
_PER_W = N_ELEM // _SC_NS
_CHUNK = 16384
_UNROLL = 8
_NB = 4096

_sc_mesh = plsc.VectorSubcoreMesh(core_axis_name="c", subcore_axis_name="s")


@functools.partial(
    pl.kernel,
    mesh=_sc_mesh,
    out_type=jax.ShapeDtypeStruct((16,), jnp.int32),
    scratch_types=[
        pltpu.VMEM((_CHUNK,), jnp.int32),
        pltpu.VMEM((_CHUNK,), jnp.int32),
        pltpu.VMEM((_SC_L, _NB), jnp.int32),
        pltpu.VMEM((1, _NB), jnp.int32),
        pltpu.VMEM((_NB,), jnp.int32),
        pltpu.VMEM((16,), jnp.int32),
        pltpu.VMEM_SHARED((_SC_NS, _NB), jnp.int32),
        pltpu.SemaphoreType.DMA,
        pltpu.SemaphoreType.DMA,
    ],
    compiler_params=pltpu.CompilerParams(needs_layout_passes=False),
)
def _sc_thresh(k_hbm, v_hbm, out_hbm,
               buf_a, buf_b, hist_v, res_v, scan_v, tb_v, sh_hist,
               sem_a, sem_b):
    core = lax.axis_index("c")
    sid = lax.axis_index("s")
    base = sid * _PER_W
    lanes = lax.iota(jnp.int32, _SC_L)
    ones = jnp.ones((_SC_L,), jnp.int32)
    zeros16 = jnp.zeros((_SC_L,), jnp.int32)
    signmask = jnp.full((_SC_L,), 0x7FFFFFFF, jnp.int32)
    iota16 = lax.iota(jnp.int32, 16)

    def _hist_pass(data_hbm, fs, fv, bs, bm):
        fs = jnp.full((_SC_L,), fs, jnp.int32)
        fv = jnp.full((_SC_L,), fv, jnp.int32)
        bs = jnp.full((_SC_L,), bs, jnp.int32)
        bm = jnp.full((_SC_L,), bm, jnp.int32)
        for r in range(_SC_L):
            @plsc.parallel_loop(0, _NB // 16, unroll=8)
            def _zero(j, r=r):
                hist_v[r, pl.ds(j * 16, 16)] = zeros16

        n_chunks = _PER_W // _CHUNK
        bufs = (buf_a, buf_b)
        sems = (sem_a, sem_b)
        handles = [pltpu.async_copy(
            data_hbm.at[pl.ds(base, _CHUNK)], buf_a, sem_a)]
        for c in range(n_chunks):
            if c + 1 < n_chunks:
                handles.append(pltpu.async_copy(
                    data_hbm.at[pl.ds(base + (c + 1) * _CHUNK, _CHUNK)],
                    bufs[(c + 1) % 2], sems[(c + 1) % 2]))
            handles[c].wait()
            buf_v = bufs[c % 2]

            @plsc.parallel_loop(0, _CHUNK // _SC_L, unroll=_UNROLL)
            def _vec(i, buf_v=buf_v):
                key = buf_v[pl.ds(i * _SC_L, _SC_L)] & signmask
                keep = lax.shift_right_logical(key, fs) == fv
                bucket = lax.shift_right_logical(key, bs) & bm
                plsc.addupdate_scatter(hist_v, [lanes, bucket], ones, mask=keep)

        @plsc.parallel_loop(0, _NB // 16, unroll=2)
        def _reduce(j):
            acc = hist_v[0, pl.ds(j * 16, 16)]
            for l in range(1, _SC_L):
                acc = acc + hist_v[l, pl.ds(j * 16, 16)]
            res_v[0, pl.ds(j * 16, 16)] = acc

        pltpu.sync_copy(res_v, sh_hist.at[pl.ds(sid, 1)])
        plsc.subcore_barrier()
        pltpu.sync_copy(sh_hist, hist_v)
        plsc.subcore_barrier()

        @plsc.parallel_loop(0, _NB // 16, unroll=2)
        def _combine(j):
            acc = hist_v[0, pl.ds(j * 16, 16)]
            for l in range(1, _SC_NS):
                acc = acc + hist_v[l, pl.ds(j * 16, 16)]
            scan_v[pl.ds(j * 16, 16)] = acc

    def _boundary(rank):
        def body(j, carry):
            run, bestg, babove = carry
            g = (_NB // 16 - 1) - j
            gsum = jnp.sum(scan_v[pl.ds(g * 16, 16)])
            newrun = run + gsum
            hit = (bestg < 0) & (newrun >= rank)
            bestg = jnp.where(hit, g, bestg)
            babove = jnp.where(hit, run, babove)
            return newrun, bestg, babove

        _, bg, babove = lax.fori_loop(
            0, _NB // 16, body,
            (jnp.int32(0), jnp.int32(-1), jnp.int32(0)))
        vec = scan_v[pl.ds(bg * 16, 16)]
        rc = lax.rev(jnp.cumsum(lax.rev(vec, (0,))), (0,))
        rr = rank - babove
        ii = jnp.max(jnp.where(rc >= rr, iota16, 0))
        rcii = jnp.max(jnp.where(iota16 == ii, rc, 0))
        vii = jnp.max(jnp.where(iota16 == ii, vec, 0))
        b = bg * 16 + ii
        rank2 = rank - (babove + rcii - vii)
        return b, rank2

    def _phase(data_hbm, out_off):
        _hist_pass(data_hbm, 31, 0, 19, _NB - 1)
        b1, rank2 = _boundary(jnp.int32(N_OUT))
        plsc.subcore_barrier()
        _hist_pass(data_hbm, 19, b1, 8, 2047)
        b2, _ = _boundary(rank2)

        @pl.when(sid == 0)
        def _():
            tb_v[...] = jnp.broadcast_to((b1 << 19) | (b2 << 8), (16,))
            pltpu.sync_copy(tb_v.at[pl.ds(0, 8)],
                            out_hbm.at[pl.ds(out_off, 8)])

    @pl.when(core == 0)
    def _():
        _phase(k_hbm, 0)

    @pl.when(core == 1)
    def _():
        _phase(v_hbm, 8)


def _thresholds(k, v):
    kf = lax.bitcast_convert_type(k.reshape(-1), jnp.int32)
    vf = lax.bitcast_convert_type(v.reshape(-1), jnp.int32)
    tb = _sc_thresh(kf, vf)
    t = lax.bitcast_convert_type(tb, jnp.float32)
    return t[0], t[8]


def _scale_kernel(k_ref, v_ref, thr_ref, kmax_ref, vmax_ref):
    h = pl.program_id(0)
    kabs = jnp.abs(k_ref[0, 0])
    vabs = jnp.abs(v_ref[0, 0])
    t_k = thr_ref[0:1, :]
    t_v = thr_ref[1:2, :]
    km = jnp.where(kabs < t_k, kabs, 0.0)
    rows = lax.broadcasted_iota(jnp.int32, (S, 1), 0)
    vmask = (vabs < t_v) & (rows >= SINK_LENGTH)
    vm = jnp.where(vmask, vabs, 0.0)
    kblk = jnp.max(km, axis=0, keepdims=True)
    vblk = jnp.max(vm, axis=0, keepdims=True)

    @pl.when(h == 0)
    def _():
        kmax_ref[...] = jnp.zeros_like(kmax_ref)
        vmax_ref[...] = jnp.zeros_like(vmax_ref)

    kmax_ref[...] = jnp.maximum(kmax_ref[...], kblk)
    vmax_ref[...] = jnp.maximum(vmax_ref[...], vblk)


def _masked_absmax(k, v, thr):
    out = pl.pallas_call(
        _scale_kernel,
        grid=(H,),
        in_specs=[
            pl.BlockSpec((1, 1, S, D), lambda h: (0, h, 0, 0)),
            pl.BlockSpec((1, 1, S, D), lambda h: (0, h, 0, 0)),
            pl.BlockSpec((8, D), lambda h: (0, 0)),
        ],
        out_specs=[
            pl.BlockSpec((1, D), lambda h: (0, 0)),
            pl.BlockSpec((1, D), lambda h: (0, 0)),
        ],
        out_shape=[
            jax.ShapeDtypeStruct((1, D), jnp.float32),
            jax.ShapeDtypeStruct((1, D), jnp.float32),
        ],
    )(k, v, thr)
    return out


def _flash_kernel(params_ref, q_ref, k_ref, v_ref, o_ref, krec_ref, vrec_ref):
    qb = pl.program_id(1)

    @pl.when(qb == 0)
    def _():
        kraw = k_ref[0, 0]
        vraw = v_ref[0, 0]
        ks = params_ref[0:1, :]
        vs = params_ref[1:2, :]
        t_k = params_ref[2:3, :]
        t_v = params_ref[3:4, :]
        kdq = jnp.clip(jnp.round(kraw / ks), -QMAX, QMAX) * ks
        krec_ref[...] = jnp.where(jnp.abs(kraw) >= t_k, kraw, kdq)
        vdq = jnp.clip(jnp.round(vraw / vs), -QMAX, QMAX) * vs
        rows = lax.broadcasted_iota(jnp.int32, (S, 1), 0)
        keep = (jnp.abs(vraw) >= t_v) | (rows < SINK_LENGTH)
        vrec_ref[...] = jnp.where(keep, vraw, vdq)

    qblk = q_ref[0, 0] * (1.0 / math.sqrt(float(D)))
    n_chunks = 4
    half = S // n_chunks
    os_, ms_, ls_ = [], [], []
    for c in range(n_chunks):
        s = lax.dot_general(
            qblk, krec_ref[pl.ds(c * half, half), :], (((1,), (1,)), ((), ())),
            preferred_element_type=jnp.float32,
            precision=lax.Precision.DEFAULT,
        )
        m = jnp.max(s, axis=-1, keepdims=True)
        p = jnp.exp(s - m)
        l = jnp.sum(p, axis=-1, keepdims=True)
        o = lax.dot_general(
            p, vrec_ref[pl.ds(c * half, half), :], (((1,), (0,)), ((), ())),
            preferred_element_type=jnp.float32,
            precision=lax.Precision.DEFAULT,
        )
        os_.append(o); ms_.append(m); ls_.append(l)
    mm = functools.reduce(jnp.maximum, ms_)
    onum = None
    oden = None
    for c in range(n_chunks):
        a = jnp.exp(ms_[c] - mm)
        onum = os_[c] * a if onum is None else onum + os_[c] * a
        oden = ls_[c] * a if oden is None else oden + ls_[c] * a
    o_ref[0, 0] = onum / oden


def _attention(params, q, k, v):
    return pl.pallas_call(
        _flash_kernel,
        grid=(H, S // BLK_Q),
        in_specs=[
            pl.BlockSpec((8, D), lambda h, qb: (0, 0)),
            pl.BlockSpec((1, 1, BLK_Q, D), lambda h, qb: (0, h, qb, 0)),
            pl.BlockSpec((1, 1, S, D), lambda h, qb: (0, h, 0, 0)),
            pl.BlockSpec((1, 1, S, D), lambda h, qb: (0, h, 0, 0)),
        ],
        out_specs=pl.BlockSpec((1, 1, BLK_Q, D), lambda h, qb: (0, h, qb, 0)),
        out_shape=jax.ShapeDtypeStruct((B, H, S, D), jnp.float32),
        scratch_shapes=[
            pltpu.VMEM((S, D), jnp.float32),
            pltpu.VMEM((S, D), jnp.float32),
        ],
    )(params, q, k, v)


def kernel(q_tensor, k_tensor, v_tensor):
    t_k, t_v = _thresholds(k_tensor, v_tensor)

    thr = jnp.zeros((8, D), jnp.float32)
    thr = thr.at[0, :].set(t_k)
    thr = thr.at[1, :].set(t_v)

    kmax, vmax = _masked_absmax(k_tensor, v_tensor, thr)
    k_scale = jnp.maximum(kmax[0], 1e-6) / QMAX
    v_scale = jnp.maximum(vmax[0], 1e-6) / QMAX

    params = jnp.zeros((8, D), jnp.float32)
    params = params.at[0, :].set(k_scale)
    params = params.at[1, :].set(v_scale)
    params = params.at[2, :].set(t_k)
    params = params.at[3, :].set(t_v)

    return _attention(params, q_tensor, k_tensor, v_tensor)


# speedup vs baseline: 40.4137x; 1.0130x over previous
import functools
import math

import jax
import jax.numpy as jnp
from jax import lax
from jax.experimental import pallas as pl
from jax.experimental.pallas import tpu as pltpu
from jax.experimental.pallas import tpu_sc as plsc

B, H, S, D = 1, 16, 2048, 128
N_ELEM = B * H * S * D
N_OUT = int(0.005 * N_ELEM)
SINK_LENGTH = 4
QMAX = 127.0
BLK_Q = 512


_SC_NC, _SC_NS, _SC_L = 2, 16, 16
_PER_W = N_ELEM // _SC_NS
_CHUNK = 16384
_UNROLL = 8
_NB = 4096

_sc_mesh = plsc.VectorSubcoreMesh(core_axis_name="c", subcore_axis_name="s")


@functools.partial(
    pl.kernel,
    mesh=_sc_mesh,
    out_type=jax.ShapeDtypeStruct((16,), jnp.int32),
    scratch_types=[
        pltpu.VMEM((_CHUNK,), jnp.int32),
        pltpu.VMEM((_CHUNK,), jnp.int32),
        pltpu.VMEM((_SC_L, _NB), jnp.int32),
        pltpu.VMEM((1, _NB), jnp.int32),
        pltpu.VMEM((_NB,), jnp.int32),
        pltpu.VMEM((16,), jnp.int32),
        pltpu.VMEM_SHARED((_SC_NS, _NB), jnp.int32),
        pltpu.SemaphoreType.DMA,
        pltpu.SemaphoreType.DMA,
    ],
    compiler_params=pltpu.CompilerParams(needs_layout_passes=False),
)
def _sc_thresh(k_hbm, v_hbm, out_hbm,
               buf_a, buf_b, hist_v, res_v, scan_v, tb_v, sh_hist,
               sem_a, sem_b):
    core = lax.axis_index("c")
    sid = lax.axis_index("s")
    base = sid * _PER_W
    lanes = lax.iota(jnp.int32, _SC_L)
    ones = jnp.ones((_SC_L,), jnp.int32)
    zeros16 = jnp.zeros((_SC_L,), jnp.int32)
    signmask = jnp.full((_SC_L,), 0x7FFFFFFF, jnp.int32)
    iota16 = lax.iota(jnp.int32, 16)

    def _hist_pass(data_hbm, fs, fv, bs, bm):
        fs = jnp.full((_SC_L,), fs, jnp.int32)
        fv = jnp.full((_SC_L,), fv, jnp.int32)
        bs = jnp.full((_SC_L,), bs, jnp.int32)
        bm = jnp.full((_SC_L,), bm, jnp.int32)
        for r in range(_SC_L):
            @plsc.parallel_loop(0, _NB // 16, unroll=8)
            def _zero(j, r=r):
                hist_v[r, pl.ds(j * 16, 16)] = zeros16

        n_chunks = _PER_W // _CHUNK
        bufs = (buf_a, buf_b)
        sems = (sem_a, sem_b)
        handles = [pltpu.async_copy(
            data_hbm.at[pl.ds(base, _CHUNK)], buf_a, sem_a)]
        for c in range(n_chunks):
            if c + 1 < n_chunks:
                handles.append(pltpu.async_copy(
                    data_hbm.at[pl.ds(base + (c + 1) * _CHUNK, _CHUNK)],
                    bufs[(c + 1) % 2], sems[(c + 1) % 2]))
            handles[c].wait()
            buf_v = bufs[c % 2]

            @plsc.parallel_loop(0, _CHUNK // _SC_L, unroll=_UNROLL)
            def _vec(i, buf_v=buf_v):
                key = buf_v[pl.ds(i * _SC_L, _SC_L)] & signmask
                keep = lax.shift_right_logical(key, fs) == fv
                bucket = lax.shift_right_logical(key, bs) & bm
                plsc.addupdate_scatter(hist_v, [lanes, bucket], ones, mask=keep)

        @plsc.parallel_loop(0, _NB // 16, unroll=2)
        def _reduce(j):
            acc = hist_v[0, pl.ds(j * 16, 16)]
            for l in range(1, _SC_L):
                acc = acc + hist_v[l, pl.ds(j * 16, 16)]
            res_v[0, pl.ds(j * 16, 16)] = acc

        pltpu.sync_copy(res_v, sh_hist.at[pl.ds(sid, 1)])
        plsc.subcore_barrier()
        pltpu.sync_copy(sh_hist, hist_v)
        plsc.subcore_barrier()

        @plsc.parallel_loop(0, _NB // 16, unroll=2)
        def _combine(j):
            acc = hist_v[0, pl.ds(j * 16, 16)]
            for l in range(1, _SC_NS):
                acc = acc + hist_v[l, pl.ds(j * 16, 16)]
            scan_v[pl.ds(j * 16, 16)] = acc

    def _boundary(rank):
        def body(j, carry):
            run, bestg, babove = carry
            g = (_NB // 16 - 1) - j
            gsum = jnp.sum(scan_v[pl.ds(g * 16, 16)])
            newrun = run + gsum
            hit = (bestg < 0) & (newrun >= rank)
            bestg = jnp.where(hit, g, bestg)
            babove = jnp.where(hit, run, babove)
            return newrun, bestg, babove

        _, bg, babove = lax.fori_loop(
            0, _NB // 16, body,
            (jnp.int32(0), jnp.int32(-1), jnp.int32(0)))
        vec = scan_v[pl.ds(bg * 16, 16)]
        rc = lax.rev(jnp.cumsum(lax.rev(vec, (0,))), (0,))
        rr = rank - babove
        ii = jnp.max(jnp.where(rc >= rr, iota16, 0))
        rcii = jnp.max(jnp.where(iota16 == ii, rc, 0))
        vii = jnp.max(jnp.where(iota16 == ii, vec, 0))
        b = bg * 16 + ii
        rank2 = rank - (babove + rcii - vii)
        return b, rank2

    def _phase(data_hbm, out_off):
        _hist_pass(data_hbm, 31, 0, 19, _NB - 1)
        b1, rank2 = _boundary(jnp.int32(N_OUT))
        plsc.subcore_barrier()
        _hist_pass(data_hbm, 19, b1, 8, 2047)
        b2, _ = _boundary(rank2)

        @pl.when(sid == 0)
        def _():
            tb_v[...] = jnp.broadcast_to((b1 << 19) | (b2 << 8), (16,))
            pltpu.sync_copy(tb_v.at[pl.ds(0, 8)],
                            out_hbm.at[pl.ds(out_off, 8)])

    @pl.when(core == 0)
    def _():
        _phase(k_hbm, 0)

    @pl.when(core == 1)
    def _():
        _phase(v_hbm, 8)


def _scale_kernel(k_ref, v_ref, tb_ref, par_ref):
    h = pl.program_id(0)
    t_k = lax.bitcast_convert_type(tb_ref[0, 0], jnp.float32)
    t_v = lax.bitcast_convert_type(tb_ref[0, 8], jnp.float32)
    kabs = jnp.abs(k_ref[0, 0])
    vabs = jnp.abs(v_ref[0, 0])
    km = jnp.where(kabs < t_k, kabs, 0.0)
    rows = lax.broadcasted_iota(jnp.int32, (S, 1), 0)
    vmask = (vabs < t_v) & (rows >= SINK_LENGTH)
    vm = jnp.where(vmask, vabs, 0.0)
    kblk = jnp.max(km, axis=0, keepdims=True)
    vblk = jnp.max(vm, axis=0, keepdims=True)

    @pl.when(h == 0)
    def _():
        par_ref[...] = jnp.zeros_like(par_ref)

    par_ref[0:1, :] = jnp.maximum(par_ref[0:1, :], kblk)
    par_ref[1:2, :] = jnp.maximum(par_ref[1:2, :], vblk)

    @pl.when(h == H - 1)
    def _():
        par_ref[0:1, :] = jnp.maximum(par_ref[0:1, :], 1e-6) / QMAX
        par_ref[1:2, :] = jnp.maximum(par_ref[1:2, :], 1e-6) / QMAX
        par_ref[2:3, :] = jnp.full((1, D), t_k)
        par_ref[3:4, :] = jnp.full((1, D), t_v)


def _params_block(k, v, tb):
    return pl.pallas_call(
        _scale_kernel,
        grid=(H,),
        in_specs=[
            pl.BlockSpec((1, 1, S, D), lambda h: (0, h, 0, 0)),
            pl.BlockSpec((1, 1, S, D), lambda h: (0, h, 0, 0)),
            pl.BlockSpec((1, 16), lambda h: (0, 0)),
        ],
        out_specs=pl.BlockSpec((8, D), lambda h: (0, 0)),
        out_shape=jax.ShapeDtypeStruct((8, D), jnp.float32),
    )(k, v, tb)


def _flash_kernel(params_ref, q_ref, k_ref, v_ref, o_ref, krec_ref, vrec_ref):
    qb = pl.program_id(1)

    @pl.when(qb == 0)
    def _():
        kraw = k_ref[0, 0]
        vraw = v_ref[0, 0]
        ks = params_ref[0:1, :]
        vs = params_ref[1:2, :]
        t_k = params_ref[2:3, :]
        t_v = params_ref[3:4, :]
        kdq = jnp.clip(jnp.round(kraw / ks), -QMAX, QMAX) * ks
        krec_ref[...] = jnp.where(jnp.abs(kraw) >= t_k, kraw, kdq)
        vdq = jnp.clip(jnp.round(vraw / vs), -QMAX, QMAX) * vs
        rows = lax.broadcasted_iota(jnp.int32, (S, 1), 0)
        keep = (jnp.abs(vraw) >= t_v) | (rows < SINK_LENGTH)
        vrec_ref[...] = jnp.where(keep, vraw, vdq)

    qblk = q_ref[0, 0] * (1.0 / math.sqrt(float(D)))
    n_chunks = 4
    half = S // n_chunks
    os_, ms_, ls_ = [], [], []
    for c in range(n_chunks):
        s = lax.dot_general(
            qblk, krec_ref[pl.ds(c * half, half), :], (((1,), (1,)), ((), ())),
            preferred_element_type=jnp.float32,
            precision=lax.Precision.DEFAULT,
        )
        m = jnp.max(s, axis=-1, keepdims=True)
        p = jnp.exp(s - m)
        l = jnp.sum(p, axis=-1, keepdims=True)
        o = lax.dot_general(
            p, vrec_ref[pl.ds(c * half, half), :], (((1,), (0,)), ((), ())),
            preferred_element_type=jnp.float32,
            precision=lax.Precision.DEFAULT,
        )
        os_.append(o); ms_.append(m); ls_.append(l)
    mm = functools.reduce(jnp.maximum, ms_)
    onum = None
    oden = None
    for c in range(n_chunks):
        a = jnp.exp(ms_[c] - mm)
        onum = os_[c] * a if onum is None else onum + os_[c] * a
        oden = ls_[c] * a if oden is None else oden + ls_[c] * a
    o_ref[0, 0] = onum / oden


def _attention(params, q, k, v):
    return pl.pallas_call(
        _flash_kernel,
        grid=(H, S // BLK_Q),
        in_specs=[
            pl.BlockSpec((8, D), lambda h, qb: (0, 0)),
            pl.BlockSpec((1, 1, BLK_Q, D), lambda h, qb: (0, h, qb, 0)),
            pl.BlockSpec((1, 1, S, D), lambda h, qb: (0, h, 0, 0)),
            pl.BlockSpec((1, 1, S, D), lambda h, qb: (0, h, 0, 0)),
        ],
        out_specs=pl.BlockSpec((1, 1, BLK_Q, D), lambda h, qb: (0, h, qb, 0)),
        out_shape=jax.ShapeDtypeStruct((B, H, S, D), jnp.float32),
        scratch_shapes=[
            pltpu.VMEM((S, D), jnp.float32),
            pltpu.VMEM((S, D), jnp.float32),
        ],
    )(params, q, k, v)


def kernel(q_tensor, k_tensor, v_tensor):
    kf = lax.bitcast_convert_type(k_tensor.reshape(-1), jnp.int32)
    vf = lax.bitcast_convert_type(v_tensor.reshape(-1), jnp.int32)
    tb = _sc_thresh(kf, vf).reshape(1, 16)
    params = _params_block(k_tensor, v_tensor, tb)
    return _attention(params, q_tensor, k_tensor, v_tensor)


# speedup vs baseline: 41.8005x; 1.0343x over previous
import functools
import math

import jax
import jax.numpy as jnp
from jax import lax
from jax.experimental import pallas as pl
from jax.experimental.pallas import tpu as pltpu
from jax.experimental.pallas import tpu_sc as plsc

B, H, S, D = 1, 16, 2048, 128
N_ELEM = B * H * S * D
N_OUT = int(0.005 * N_ELEM)
SINK_LENGTH = 4
QMAX = 127.0
BLK_Q = 1024


_SC_NC, _SC_NS, _SC_L = 2, 16, 16
_PER_W = N_ELEM // _SC_NS
_CHUNK = 16384
_UNROLL = 8
_NB = 4096

_sc_mesh = plsc.VectorSubcoreMesh(core_axis_name="c", subcore_axis_name="s")


@functools.partial(
    pl.kernel,
    mesh=_sc_mesh,
    out_type=jax.ShapeDtypeStruct((16,), jnp.int32),
    scratch_types=[
        pltpu.VMEM((_CHUNK,), jnp.int32),
        pltpu.VMEM((_CHUNK,), jnp.int32),
        pltpu.VMEM((_SC_L, _NB), jnp.int32),
        pltpu.VMEM((1, _NB), jnp.int32),
        pltpu.VMEM((_NB,), jnp.int32),
        pltpu.VMEM((16,), jnp.int32),
        pltpu.VMEM_SHARED((_SC_NS, _NB), jnp.int32),
        pltpu.SemaphoreType.DMA,
        pltpu.SemaphoreType.DMA,
    ],
    compiler_params=pltpu.CompilerParams(needs_layout_passes=False),
)
def _sc_thresh(k_hbm, v_hbm, out_hbm,
               buf_a, buf_b, hist_v, res_v, scan_v, tb_v, sh_hist,
               sem_a, sem_b):
    core = lax.axis_index("c")
    sid = lax.axis_index("s")
    base = sid * _PER_W
    lanes = lax.iota(jnp.int32, _SC_L)
    ones = jnp.ones((_SC_L,), jnp.int32)
    zeros16 = jnp.zeros((_SC_L,), jnp.int32)
    signmask = jnp.full((_SC_L,), 0x7FFFFFFF, jnp.int32)
    iota16 = lax.iota(jnp.int32, 16)

    def _hist_pass(data_hbm, fs, fv, bs, bm):
        fs = jnp.full((_SC_L,), fs, jnp.int32)
        fv = jnp.full((_SC_L,), fv, jnp.int32)
        bs = jnp.full((_SC_L,), bs, jnp.int32)
        bm = jnp.full((_SC_L,), bm, jnp.int32)
        for r in range(_SC_L):
            @plsc.parallel_loop(0, _NB // 16, unroll=8)
            def _zero(j, r=r):
                hist_v[r, pl.ds(j * 16, 16)] = zeros16

        n_chunks = _PER_W // _CHUNK
        bufs = (buf_a, buf_b)
        sems = (sem_a, sem_b)
        handles = [pltpu.async_copy(
            data_hbm.at[pl.ds(base, _CHUNK)], buf_a, sem_a)]
        for c in range(n_chunks):
            if c + 1 < n_chunks:
                handles.append(pltpu.async_copy(
                    data_hbm.at[pl.ds(base + (c + 1) * _CHUNK, _CHUNK)],
                    bufs[(c + 1) % 2], sems[(c + 1) % 2]))
            handles[c].wait()
            buf_v = bufs[c % 2]

            @plsc.parallel_loop(0, _CHUNK // _SC_L, unroll=_UNROLL)
            def _vec(i, buf_v=buf_v):
                key = buf_v[pl.ds(i * _SC_L, _SC_L)] & signmask
                keep = lax.shift_right_logical(key, fs) == fv
                bucket = lax.shift_right_logical(key, bs) & bm
                plsc.addupdate_scatter(hist_v, [lanes, bucket], ones, mask=keep)

        @plsc.parallel_loop(0, _NB // 16, unroll=2)
        def _reduce(j):
            acc = hist_v[0, pl.ds(j * 16, 16)]
            for l in range(1, _SC_L):
                acc = acc + hist_v[l, pl.ds(j * 16, 16)]
            res_v[0, pl.ds(j * 16, 16)] = acc

        pltpu.sync_copy(res_v, sh_hist.at[pl.ds(sid, 1)])
        plsc.subcore_barrier()
        pltpu.sync_copy(sh_hist, hist_v)
        plsc.subcore_barrier()

        @plsc.parallel_loop(0, _NB // 16, unroll=2)
        def _combine(j):
            acc = hist_v[0, pl.ds(j * 16, 16)]
            for l in range(1, _SC_NS):
                acc = acc + hist_v[l, pl.ds(j * 16, 16)]
            scan_v[pl.ds(j * 16, 16)] = acc

    def _boundary(rank):
        def body(j, carry):
            run, bestg, babove = carry
            g = (_NB // 16 - 1) - j
            gsum = jnp.sum(scan_v[pl.ds(g * 16, 16)])
            newrun = run + gsum
            hit = (bestg < 0) & (newrun >= rank)
            bestg = jnp.where(hit, g, bestg)
            babove = jnp.where(hit, run, babove)
            return newrun, bestg, babove

        _, bg, babove = lax.fori_loop(
            0, _NB // 16, body,
            (jnp.int32(0), jnp.int32(-1), jnp.int32(0)))
        vec = scan_v[pl.ds(bg * 16, 16)]
        rc = lax.rev(jnp.cumsum(lax.rev(vec, (0,))), (0,))
        rr = rank - babove
        ii = jnp.max(jnp.where(rc >= rr, iota16, 0))
        rcii = jnp.max(jnp.where(iota16 == ii, rc, 0))
        vii = jnp.max(jnp.where(iota16 == ii, vec, 0))
        b = bg * 16 + ii
        rank2 = rank - (babove + rcii - vii)
        return b, rank2

    def _phase(data_hbm, out_off):
        _hist_pass(data_hbm, 31, 0, 19, _NB - 1)
        b1, rank2 = _boundary(jnp.int32(N_OUT))
        plsc.subcore_barrier()
        _hist_pass(data_hbm, 19, b1, 8, 2047)
        b2, _ = _boundary(rank2)

        @pl.when(sid == 0)
        def _():
            tb_v[...] = jnp.broadcast_to((b1 << 19) | (b2 << 8), (16,))
            pltpu.sync_copy(tb_v.at[pl.ds(0, 8)],
                            out_hbm.at[pl.ds(out_off, 8)])

    @pl.when(core == 0)
    def _():
        _phase(k_hbm, 0)

    @pl.when(core == 1)
    def _():
        _phase(v_hbm, 8)


def _scale_kernel(k_ref, v_ref, tb_ref, par_ref):
    h = pl.program_id(0)
    t_k = lax.bitcast_convert_type(tb_ref[0, 0], jnp.float32)
    t_v = lax.bitcast_convert_type(tb_ref[0, 8], jnp.float32)
    kabs = jnp.abs(k_ref[0, 0])
    vabs = jnp.abs(v_ref[0, 0])
    km = jnp.where(kabs < t_k, kabs, 0.0)
    rows = lax.broadcasted_iota(jnp.int32, (S, 1), 0)
    vmask = (vabs < t_v) & (rows >= SINK_LENGTH)
    vm = jnp.where(vmask, vabs, 0.0)
    kblk = jnp.max(km, axis=0, keepdims=True)
    vblk = jnp.max(vm, axis=0, keepdims=True)

    @pl.when(h == 0)
    def _():
        par_ref[...] = jnp.zeros_like(par_ref)

    par_ref[0:1, :] = jnp.maximum(par_ref[0:1, :], kblk)
    par_ref[1:2, :] = jnp.maximum(par_ref[1:2, :], vblk)

    @pl.when(h == H - 1)
    def _():
        par_ref[0:1, :] = jnp.maximum(par_ref[0:1, :], 1e-6) / QMAX
        par_ref[1:2, :] = jnp.maximum(par_ref[1:2, :], 1e-6) / QMAX
        par_ref[2:3, :] = jnp.full((1, D), t_k)
        par_ref[3:4, :] = jnp.full((1, D), t_v)


def _params_block(k, v, tb):
    return pl.pallas_call(
        _scale_kernel,
        grid=(H,),
        in_specs=[
            pl.BlockSpec((1, 1, S, D), lambda h: (0, h, 0, 0)),
            pl.BlockSpec((1, 1, S, D), lambda h: (0, h, 0, 0)),
            pl.BlockSpec((1, 16), lambda h: (0, 0)),
        ],
        out_specs=pl.BlockSpec((8, D), lambda h: (0, 0)),
        out_shape=jax.ShapeDtypeStruct((8, D), jnp.float32),
    )(k, v, tb)


def _flash_kernel(params_ref, q_ref, k_ref, v_ref, o_ref, krec_ref, vrec_ref):
    qb = pl.program_id(1)

    @pl.when(qb == 0)
    def _():
        kraw = k_ref[0, 0]
        vraw = v_ref[0, 0]
        ks = params_ref[0:1, :]
        vs = params_ref[1:2, :]
        t_k = params_ref[2:3, :]
        t_v = params_ref[3:4, :]
        kdq = jnp.clip(jnp.round(kraw / ks), -QMAX, QMAX) * ks
        krec_ref[...] = jnp.where(jnp.abs(kraw) >= t_k, kraw, kdq)
        vdq = jnp.clip(jnp.round(vraw / vs), -QMAX, QMAX) * vs
        rows = lax.broadcasted_iota(jnp.int32, (S, 1), 0)
        keep = (jnp.abs(vraw) >= t_v) | (rows < SINK_LENGTH)
        vrec_ref[...] = jnp.where(keep, vraw, vdq)

    qblk = q_ref[0, 0] * (1.0 / math.sqrt(float(D)))
    n_chunks = 4
    half = S // n_chunks
    os_, ms_, ls_ = [], [], []
    for c in range(n_chunks):
        s = lax.dot_general(
            qblk, krec_ref[pl.ds(c * half, half), :], (((1,), (1,)), ((), ())),
            preferred_element_type=jnp.float32,
            precision=lax.Precision.DEFAULT,
        )
        m = jnp.max(s, axis=-1, keepdims=True)
        p = jnp.exp(s - m)
        l = jnp.sum(p, axis=-1, keepdims=True)
        o = lax.dot_general(
            p, vrec_ref[pl.ds(c * half, half), :], (((1,), (0,)), ((), ())),
            preferred_element_type=jnp.float32,
            precision=lax.Precision.DEFAULT,
        )
        os_.append(o); ms_.append(m); ls_.append(l)
    mm = functools.reduce(jnp.maximum, ms_)
    onum = None
    oden = None
    for c in range(n_chunks):
        a = jnp.exp(ms_[c] - mm)
        onum = os_[c] * a if onum is None else onum + os_[c] * a
        oden = ls_[c] * a if oden is None else oden + ls_[c] * a
    o_ref[0, 0] = onum / oden


def _attention(params, q, k, v):
    return pl.pallas_call(
        _flash_kernel,
        grid=(H, S // BLK_Q),
        in_specs=[
            pl.BlockSpec((8, D), lambda h, qb: (0, 0)),
            pl.BlockSpec((1, 1, BLK_Q, D), lambda h, qb: (0, h, qb, 0)),
            pl.BlockSpec((1, 1, S, D), lambda h, qb: (0, h, 0, 0)),
            pl.BlockSpec((1, 1, S, D), lambda h, qb: (0, h, 0, 0)),
        ],
        out_specs=pl.BlockSpec((1, 1, BLK_Q, D), lambda h, qb: (0, h, qb, 0)),
        out_shape=jax.ShapeDtypeStruct((B, H, S, D), jnp.float32),
        scratch_shapes=[
            pltpu.VMEM((S, D), jnp.float32),
            pltpu.VMEM((S, D), jnp.float32),
        ],
    )(params, q, k, v)


def kernel(q_tensor, k_tensor, v_tensor):
    kf = lax.bitcast_convert_type(k_tensor.reshape(-1), jnp.int32)
    vf = lax.bitcast_convert_type(v_tensor.reshape(-1), jnp.int32)
    tb = _sc_thresh(kf, vf).reshape(1, 16)
    params = _params_block(k_tensor, v_tensor, tb)
    return _attention(params, q_tensor, k_tensor, v_tensor)
